# baseline (speedup 1.0000x reference)
import functools

import jax
import jax.numpy as jnp
from jax import lax
from jax.experimental import pallas as pl
from jax.experimental.pallas import tpu as pltpu
from jax.experimental.pallas import tpu_sc as plsc

_NC = 2
_NS = 16
_L = 16
_K = 128


def _sc_mesh():
    return plsc.VectorSubcoreMesh(core_axis_name="c", subcore_axis_name="s",
                                  num_cores=_NC, num_subcores=_NS)


def _deg_counts(dstb, n_pad):
    rows_per_sub = dstb.shape[0] // (_NC * _NS)
    nps = n_pad // _NS
    zrows = 100
    mesh = _sc_mesh()

    @functools.partial(
        pl.kernel,
        out_type=jax.ShapeDtypeStruct((_NC * n_pad, _L), jnp.float32),
        mesh=mesh,
        scratch_types=[
            pltpu.VMEM((rows_per_sub, _K), jnp.int32),
            pltpu.VMEM((_K, _L), jnp.float32),
            pltpu.VMEM((zrows, _L), jnp.float32),
            pltpu.VMEM_SHARED((n_pad, _L), jnp.float32),
        ],
    )
    def deg_kernel(dstb_hbm, deg_hbm, idx_v, ones_v, zero_v, acc_sh):
        c = lax.axis_index("c")
        s = lax.axis_index("s")

        @pl.loop(0, _K)
        def _(i):
            ones_v.at[i][...] = jnp.full((_L,), 1.0, jnp.float32)

        @pl.loop(0, zrows)
        def _(i):
            zero_v.at[i][...] = jnp.zeros((_L,), jnp.float32)

        @pl.loop(0, nps // zrows)
        def _(i):
            pltpu.sync_copy(zero_v, acc_sh.at[pl.ds(s * nps + i * zrows, zrows)])

        row0 = (c * _NS + s) * rows_per_sub
        pltpu.sync_copy(dstb_hbm.at[pl.ds(row0, rows_per_sub)], idx_v)
        plsc.subcore_barrier()

        @pl.loop(0, rows_per_sub)
        def _(j):
            pltpu.sync_copy(ones_v, acc_sh.at[idx_v.at[j]], add=True)

        plsc.subcore_barrier()
        pltpu.sync_copy(acc_sh.at[pl.ds(s * nps, nps)],
                        deg_hbm.at[pl.ds(c * n_pad + s * nps, nps)])

    return deg_kernel(dstb)


def _edge_aggregate(u_all, srcb, dstb, n_pad, dh):
    KE = dstb.shape[1]
    rows_per_sub = dstb.shape[0] // _NS
    n_ph = 4
    ph = rows_per_sub // n_ph
    nps = n_pad // _NS
    mesh = _sc_mesh()

    @functools.partial(
        pl.kernel,
        out_type=jax.ShapeDtypeStruct((_NC * n_pad, dh), jnp.float32),
        mesh=mesh,
        scratch_types=[
            pltpu.VMEM((ph, KE), jnp.int32),
            pltpu.VMEM((ph, KE), jnp.int32),
            [pltpu.VMEM((KE, dh), jnp.float32)] * 4,
            [pltpu.SemaphoreType.DMA] * 4,
            [pltpu.SemaphoreType.DMA] * 4,
            pltpu.VMEM_SHARED((n_pad, dh), jnp.float32),
        ],
    )
    def edge_kernel(u_hbm, srcb_hbm, dstb_hbm, agg_hbm,
                    src_v, dst_v, bufs, gsems, ssems, acc_sh):
        c = lax.axis_index("c")
        s = lax.axis_index("s")
        pltpu.sync_copy(u_hbm.at[pl.ds(c * n_pad + s * nps, nps)],
                        acc_sh.at[pl.ds(s * nps, nps)])
        plsc.subcore_barrier()
        src_base = (c * _NS + s) * rows_per_sub
        dst_base = s * rows_per_sub

        def gather(r, b):
            return pltpu.make_async_copy(u_hbm.at[src_v.at[r]], bufs[b],
                                         gsems[b])

        def scat(r, b):
            return pltpu.make_async_copy(bufs[b], acc_sh.at[dst_v.at[r]],
                                         ssems[b])

        for p in range(n_ph):
            if p > 0:
                for b in range(4):
                    scat(0, b).wait()
            pltpu.sync_copy(srcb_hbm.at[pl.ds(src_base + p * ph, ph)], src_v)
            pltpu.sync_copy(dstb_hbm.at[pl.ds(dst_base + p * ph, ph)], dst_v)
            gather(0, 0).start()
            gather(1, 1).start()

            @pl.loop(0, ph, step=4)
            def _(t):
                for i in range(4):
                    ti = t + i
                    gather(ti, i).wait()
                    pltpu.async_copy(bufs[i], acc_sh.at[dst_v.at[ti]],
                                     ssems[i], add=True)
                    bn = (i + 2) % 4

                    @pl.when((ti >= 2) & (ti + 2 < ph))
                    def _():
                        scat(ti - 2, bn).wait()

                    @pl.when(ti + 2 < ph)
                    def _():
                        gather(ti + 2, bn).start()

        for b in range(4):
            scat(0, b).wait()

        plsc.subcore_barrier()
        pltpu.sync_copy(acc_sh.at[pl.ds(s * nps, nps)],
                        agg_hbm.at[pl.ds(c * n_pad + s * nps, nps)])

    return edge_kernel(u_all, srcb, dstb)


def _matmul(x, W):
    n, din = x.shape
    dout = W.shape[1]
    RM = 2000

    def body(x_ref, w_ref, h_ref):
        h_ref[...] = jnp.dot(x_ref[...], w_ref[...],
                             preferred_element_type=jnp.float32)

    return pl.pallas_call(
        body,
        grid=(n // RM,),
        in_specs=[pl.BlockSpec((RM, din), lambda i: (i, 0)),
                  pl.BlockSpec((din, dout), lambda i: (0, 0))],
        out_specs=pl.BlockSpec((RM, dout), lambda i: (i, 0)),
        out_shape=jax.ShapeDtypeStruct((n, dout), jnp.float32),
    )(x, W)


def _scale(h, degp, n, n_pad, dout):
    dh = dout // 2
    RM = 2048
    g = -(-n // RM)
    gp = n_pad // RM

    def body(h_ref, d0_ref, d1_ref, u_ref):
        deg = d0_ref[:, 0:1] + d1_ref[:, 0:1] + 1.0
        dis = lax.rsqrt(deg)
        j = pl.program_id(1)
        u_ref[...] = h_ref[:, pl.ds(j * dh, dh)] * dis

    return pl.pallas_call(
        body,
        grid=(g, 2),
        in_specs=[pl.BlockSpec((RM, dout), lambda i, j: (i, 0)),
                  pl.BlockSpec((RM, _L), lambda i, j: (i, 0)),
                  pl.BlockSpec((RM, _L), lambda i, j: (gp + i, 0))],
        out_specs=pl.BlockSpec((RM, dh), lambda i, j: (j * gp + i, 0)),
        out_shape=jax.ShapeDtypeStruct((2 * n_pad, dh), jnp.float32),
    )(h, degp, degp)


def _finalize(agg, degp, b2, n, n_pad, dout):
    dh = dout // 2
    RM = 2048
    g = -(-n // RM)
    gp = n_pad // RM

    def body(a0_ref, a1_ref, d0_ref, d1_ref, b_ref, o_ref):
        deg = d0_ref[:, 0:1] + d1_ref[:, 0:1] + 1.0
        dis = lax.rsqrt(deg)
        o_ref[:, 0:dh] = a0_ref[...] * dis + b_ref[:, 0:dh]
        o_ref[:, dh:dout] = a1_ref[...] * dis + b_ref[:, dh:dout]

    return pl.pallas_call(
        body,
        grid=(g,),
        in_specs=[pl.BlockSpec((RM, dh), lambda i: (i, 0)),
                  pl.BlockSpec((RM, dh), lambda i: (gp + i, 0)),
                  pl.BlockSpec((RM, _L), lambda i: (i, 0)),
                  pl.BlockSpec((RM, _L), lambda i: (gp + i, 0)),
                  pl.BlockSpec((1, dout), lambda i: (0, 0))],
        out_specs=pl.BlockSpec((RM, dout), lambda i: (i, 0)),
        out_shape=jax.ShapeDtypeStruct((n, dout), jnp.float32),
    )(agg, agg, degp, degp, b2)


def kernel(x, edge_index, W, b):
    n, _ = x.shape
    dout = W.shape[1]
    e = edge_index.shape[1]
    dh = dout // 2
    src = edge_index[0]
    dst = edge_index[1]

    n_pad = -(-n // 2048) * 2048
    chunk = _NS * 8 * _K
    e_pad = -(-e // chunk) * chunk
    pad = e_pad - e
    src_p = jnp.concatenate([src, jnp.zeros((pad,), jnp.int32)])
    trash = n + jnp.arange(pad, dtype=jnp.int32) % (n_pad - n)
    dst_p = jnp.concatenate([dst, trash])

    KE = 80
    dstb = dst_p.reshape(e_pad // KE, KE)
    dstk = dst_p.reshape(e_pad // _K, _K)
    srcb = jnp.concatenate([src_p, src_p + n_pad]).reshape(
        2 * (e_pad // KE), KE)

    degp = _deg_counts(dstk, n_pad)
    h = _matmul(x, W)
    u_all = _scale(h, degp, n, n_pad, dout)
    agg = _edge_aggregate(u_all, srcb, dstb, n_pad, dh)
    return _finalize(agg, degp, b.reshape(1, dout), n, n_pad, dout)


# speedup vs baseline: 1.0462x; 1.0462x over previous
import functools

import jax
import jax.numpy as jnp
from jax import lax
from jax.experimental import pallas as pl
from jax.experimental.pallas import tpu as pltpu
from jax.experimental.pallas import tpu_sc as plsc

_NC = 2
_NS = 16
_L = 16
_K = 128


def _sc_mesh():
    return plsc.VectorSubcoreMesh(core_axis_name="c", subcore_axis_name="s",
                                  num_cores=_NC, num_subcores=_NS)


def _deg_counts(dstb, n_pad):
    rows_per_sub = dstb.shape[0] // (_NC * _NS)
    nps = n_pad // _NS
    zrows = 100
    mesh = _sc_mesh()

    @functools.partial(
        pl.kernel,
        out_type=jax.ShapeDtypeStruct((_NC * n_pad, _L), jnp.float32),
        mesh=mesh,
        scratch_types=[
            pltpu.VMEM((rows_per_sub, _K), jnp.int32),
            pltpu.VMEM((_K, _L), jnp.float32),
            pltpu.VMEM((zrows, _L), jnp.float32),
            pltpu.VMEM_SHARED((n_pad, _L), jnp.float32),
        ],
    )
    def deg_kernel(dstb_hbm, deg_hbm, idx_v, ones_v, zero_v, acc_sh):
        c = lax.axis_index("c")
        s = lax.axis_index("s")

        @pl.loop(0, _K)
        def _(i):
            ones_v.at[i][...] = jnp.full((_L,), 1.0, jnp.float32)

        @pl.loop(0, zrows)
        def _(i):
            zero_v.at[i][...] = jnp.zeros((_L,), jnp.float32)

        @pl.loop(0, nps // zrows)
        def _(i):
            pltpu.sync_copy(zero_v, acc_sh.at[pl.ds(s * nps + i * zrows, zrows)])

        row0 = (c * _NS + s) * rows_per_sub
        pltpu.sync_copy(dstb_hbm.at[pl.ds(row0, rows_per_sub)], idx_v)
        plsc.subcore_barrier()

        @pl.loop(0, rows_per_sub)
        def _(j):
            pltpu.sync_copy(ones_v, acc_sh.at[idx_v.at[j]], add=True)

        plsc.subcore_barrier()
        pltpu.sync_copy(acc_sh.at[pl.ds(s * nps, nps)],
                        deg_hbm.at[pl.ds(c * n_pad + s * nps, nps)])

    return deg_kernel(dstb)


def _edge_aggregate(u_all, srcb, dstb, n_pad, dh):
    rows_per_sub = dstb.shape[0] // _NS
    half = rows_per_sub // 2
    nps = n_pad // _NS
    mesh = _sc_mesh()

    @functools.partial(
        pl.kernel,
        out_type=jax.ShapeDtypeStruct((_NC * n_pad, dh), jnp.float32),
        mesh=mesh,
        scratch_types=[
            pltpu.VMEM((half, _K), jnp.int32),
            pltpu.VMEM((half, _K), jnp.int32),
            pltpu.VMEM((_K, dh), jnp.float32),
            pltpu.VMEM((_K, dh), jnp.float32),
            pltpu.SemaphoreType.DMA,
            pltpu.SemaphoreType.DMA,
            pltpu.VMEM_SHARED((n_pad, dh), jnp.float32),
        ],
    )
    def edge_kernel(u_hbm, srcb_hbm, dstb_hbm, agg_hbm,
                    src_v, dst_v, buf0, buf1, gsem0, gsem1, acc_sh):
        c = lax.axis_index("c")
        s = lax.axis_index("s")
        pltpu.sync_copy(u_hbm.at[pl.ds(c * n_pad + s * nps, nps)],
                        acc_sh.at[pl.ds(s * nps, nps)])
        plsc.subcore_barrier()
        src_base = (c * _NS + s) * rows_per_sub
        dst_base = s * rows_per_sub

        def gather(r, buf, sem):
            return pltpu.make_async_copy(u_hbm.at[src_v.at[r]], buf, sem)

        for p in range(2):
            pltpu.sync_copy(
                srcb_hbm.at[pl.ds(src_base + p * half, half)], src_v)
            pltpu.sync_copy(
                dstb_hbm.at[pl.ds(dst_base + p * half, half)], dst_v)
            gather(0, buf0, gsem0).start()

            @pl.loop(0, half, step=2)
            def _(t):
                gather(t + 1, buf1, gsem1).start()
                gather(t, buf0, gsem0).wait()
                pltpu.sync_copy(buf0, acc_sh.at[dst_v.at[t]], add=True)

                @pl.when(t + 2 < half)
                def _():
                    gather(t + 2, buf0, gsem0).start()

                gather(t + 1, buf1, gsem1).wait()
                pltpu.sync_copy(buf1, acc_sh.at[dst_v.at[t + 1]], add=True)

        plsc.subcore_barrier()
        pltpu.sync_copy(acc_sh.at[pl.ds(s * nps, nps)],
                        agg_hbm.at[pl.ds(c * n_pad + s * nps, nps)])

    return edge_kernel(u_all, srcb, dstb)


def _matmul(x, W):
    n, din = x.shape
    dout = W.shape[1]
    RM = 2000

    def body(x_ref, w_ref, h_ref):
        h_ref[...] = jnp.dot(x_ref[...], w_ref[...],
                             preferred_element_type=jnp.float32)

    return pl.pallas_call(
        body,
        grid=(n // RM,),
        in_specs=[pl.BlockSpec((RM, din), lambda i: (i, 0)),
                  pl.BlockSpec((din, dout), lambda i: (0, 0))],
        out_specs=pl.BlockSpec((RM, dout), lambda i: (i, 0)),
        out_shape=jax.ShapeDtypeStruct((n, dout), jnp.float32),
    )(x, W)


def _scale(h, degp, n, n_pad, dout):
    dh = dout // 2
    RM = 2048
    g = -(-n // RM)
    gp = n_pad // RM

    def body(h_ref, d0_ref, d1_ref, u_ref):
        deg = d0_ref[:, 0:1] + d1_ref[:, 0:1] + 1.0
        dis = lax.rsqrt(deg)
        j = pl.program_id(1)
        u_ref[...] = h_ref[:, pl.ds(j * dh, dh)] * dis

    return pl.pallas_call(
        body,
        grid=(g, 2),
        in_specs=[pl.BlockSpec((RM, dout), lambda i, j: (i, 0)),
                  pl.BlockSpec((RM, _L), lambda i, j: (i, 0)),
                  pl.BlockSpec((RM, _L), lambda i, j: (gp + i, 0))],
        out_specs=pl.BlockSpec((RM, dh), lambda i, j: (j * gp + i, 0)),
        out_shape=jax.ShapeDtypeStruct((2 * n_pad, dh), jnp.float32),
    )(h, degp, degp)


def _finalize(agg, degp, b2, n, n_pad, dout):
    dh = dout // 2
    RM = 2048
    g = -(-n // RM)
    gp = n_pad // RM

    def body(a0_ref, a1_ref, d0_ref, d1_ref, b_ref, o_ref):
        deg = d0_ref[:, 0:1] + d1_ref[:, 0:1] + 1.0
        dis = lax.rsqrt(deg)
        o_ref[:, 0:dh] = a0_ref[...] * dis + b_ref[:, 0:dh]
        o_ref[:, dh:dout] = a1_ref[...] * dis + b_ref[:, dh:dout]

    return pl.pallas_call(
        body,
        grid=(g,),
        in_specs=[pl.BlockSpec((RM, dh), lambda i: (i, 0)),
                  pl.BlockSpec((RM, dh), lambda i: (gp + i, 0)),
                  pl.BlockSpec((RM, _L), lambda i: (i, 0)),
                  pl.BlockSpec((RM, _L), lambda i: (gp + i, 0)),
                  pl.BlockSpec((1, dout), lambda i: (0, 0))],
        out_specs=pl.BlockSpec((RM, dout), lambda i: (i, 0)),
        out_shape=jax.ShapeDtypeStruct((n, dout), jnp.float32),
    )(agg, agg, degp, degp, b2)


def kernel(x, edge_index, W, b):
    n, _ = x.shape
    dout = W.shape[1]
    e = edge_index.shape[1]
    dh = dout // 2
    src = edge_index[0]
    dst = edge_index[1]

    n_pad = -(-n // 2048) * 2048
    chunk = _NS * 8 * _K
    e_pad = -(-e // chunk) * chunk
    pad = e_pad - e
    src_p = jnp.concatenate([src, jnp.zeros((pad,), jnp.int32)])
    trash = n + jnp.arange(pad, dtype=jnp.int32) % (n_pad - n)
    dst_p = jnp.concatenate([dst, trash])

    dstb = dst_p.reshape(e_pad // _K, _K)
    srcb = jnp.concatenate([src_p, src_p + n_pad]).reshape(
        2 * (e_pad // _K), _K)

    degp = _deg_counts(dstb, n_pad)
    h = _matmul(x, W)
    u_all = _scale(h, degp, n, n_pad, dout)
    agg = _edge_aggregate(u_all, srcb, dstb, n_pad, dh)
    return _finalize(agg, degp, b.reshape(1, dout), n, n_pad, dout)


# speedup vs baseline: 1.0495x; 1.0031x over previous
import functools

import jax
import jax.numpy as jnp
from jax import lax
from jax.experimental import pallas as pl
from jax.experimental.pallas import tpu as pltpu
from jax.experimental.pallas import tpu_sc as plsc

_NC = 2
_NS = 16
_L = 16
_K = 128


def _sc_mesh():
    return plsc.VectorSubcoreMesh(core_axis_name="c", subcore_axis_name="s",
                                  num_cores=_NC, num_subcores=_NS)


def _deg_counts(dstb, n_pad):
    rows_per_sub = dstb.shape[0] // (_NC * _NS)
    nps = n_pad // _NS
    zrows = 100
    mesh = _sc_mesh()

    @functools.partial(
        pl.kernel,
        out_type=jax.ShapeDtypeStruct((_NC * n_pad, _L), jnp.float32),
        mesh=mesh,
        scratch_types=[
            pltpu.VMEM((rows_per_sub, _K), jnp.int32),
            pltpu.VMEM((_K, _L), jnp.float32),
            pltpu.VMEM((zrows, _L), jnp.float32),
            pltpu.VMEM_SHARED((n_pad, _L), jnp.float32),
        ],
    )
    def deg_kernel(dstb_hbm, deg_hbm, idx_v, ones_v, zero_v, acc_sh):
        c = lax.axis_index("c")
        s = lax.axis_index("s")

        @pl.loop(0, _K)
        def _(i):
            ones_v.at[i][...] = jnp.full((_L,), 1.0, jnp.float32)

        @pl.loop(0, zrows)
        def _(i):
            zero_v.at[i][...] = jnp.zeros((_L,), jnp.float32)

        @pl.loop(0, nps // zrows)
        def _(i):
            pltpu.sync_copy(zero_v, acc_sh.at[pl.ds(s * nps + i * zrows, zrows)])

        row0 = (c * _NS + s) * rows_per_sub
        pltpu.sync_copy(dstb_hbm.at[pl.ds(row0, rows_per_sub)], idx_v)
        plsc.subcore_barrier()

        @pl.loop(0, rows_per_sub)
        def _(j):
            pltpu.sync_copy(ones_v, acc_sh.at[idx_v.at[j]], add=True)

        plsc.subcore_barrier()
        pltpu.sync_copy(acc_sh.at[pl.ds(s * nps, nps)],
                        deg_hbm.at[pl.ds(c * n_pad + s * nps, nps)])

    return deg_kernel(dstb)


def _edge_aggregate(u_all, srcb, dstb, n_pad, dh):
    rows_per_sub = dstb.shape[0] // _NS
    half = rows_per_sub // 2
    nps = n_pad // _NS
    mesh = _sc_mesh()

    @functools.partial(
        pl.kernel,
        out_type=jax.ShapeDtypeStruct((_NC * n_pad, dh), jnp.float32),
        mesh=mesh,
        scratch_types=[
            pltpu.VMEM((half, _K), jnp.int32),
            pltpu.VMEM((half, _K), jnp.int32),
            pltpu.VMEM((_K, dh), jnp.float32),
            pltpu.VMEM((_K, dh), jnp.float32),
            pltpu.SemaphoreType.DMA,
            pltpu.SemaphoreType.DMA,
            pltpu.VMEM_SHARED((n_pad, dh), jnp.float32),
        ],
    )
    def edge_kernel(u_hbm, srcb_hbm, dstb_hbm, agg_hbm,
                    src_v, dst_v, buf0, buf1, gsem0, gsem1, acc_sh):
        c = lax.axis_index("c")
        s = lax.axis_index("s")
        pltpu.sync_copy(u_hbm.at[pl.ds(c * n_pad + s * nps, nps)],
                        acc_sh.at[pl.ds(s * nps, nps)])
        plsc.subcore_barrier()
        src_base = (c * _NS + s) * rows_per_sub
        dst_base = s * rows_per_sub

        def gather(r, buf, sem):
            return pltpu.make_async_copy(u_hbm.at[src_v.at[r]], buf, sem)

        for p in range(2):
            pltpu.sync_copy(
                srcb_hbm.at[pl.ds(src_base + p * half, half)], src_v)
            pltpu.sync_copy(
                dstb_hbm.at[pl.ds(dst_base + p * half, half)], dst_v)
            gather(0, buf0, gsem0).start()

            @pl.loop(0, half, step=2)
            def _(t):
                gather(t + 1, buf1, gsem1).start()
                gather(t, buf0, gsem0).wait()
                pltpu.sync_copy(buf0, acc_sh.at[dst_v.at[t]], add=True)

                @pl.when(t + 2 < half)
                def _():
                    gather(t + 2, buf0, gsem0).start()

                gather(t + 1, buf1, gsem1).wait()
                pltpu.sync_copy(buf1, acc_sh.at[dst_v.at[t + 1]], add=True)

        plsc.subcore_barrier()
        pltpu.sync_copy(acc_sh.at[pl.ds(s * nps, nps)],
                        agg_hbm.at[pl.ds(c * n_pad + s * nps, nps)])

    return edge_kernel(u_all, srcb, dstb)


def _matmul(x, W):
    n, din = x.shape
    dout = W.shape[1]
    RM = 2000

    def body(x_ref, w_ref, h_ref):
        h_ref[...] = jnp.dot(x_ref[...], w_ref[...],
                             preferred_element_type=jnp.float32)

    return pl.pallas_call(
        body,
        grid=(n // RM,),
        in_specs=[pl.BlockSpec((RM, din), lambda i: (i, 0)),
                  pl.BlockSpec((din, dout), lambda i: (0, 0))],
        out_specs=pl.BlockSpec((RM, dout), lambda i: (i, 0)),
        out_shape=jax.ShapeDtypeStruct((n, dout), jnp.float32),
    )(x, W)


def _scale(h, degp, n, n_pad, dout):
    dh = dout // 2
    RM = 2048
    g = -(-n // RM)
    gp = n_pad // RM

    def body(h_ref, d0_ref, d1_ref, u_ref):
        deg = d0_ref[:, 0:1] + d1_ref[:, 0:1] + 1.0
        dis = lax.rsqrt(deg)
        u_ref[...] = h_ref[...] * dis

    return pl.pallas_call(
        body,
        grid=(g, 2),
        in_specs=[pl.BlockSpec((RM, dh), lambda i, j: (i, j)),
                  pl.BlockSpec((RM, _L), lambda i, j: (i, 0)),
                  pl.BlockSpec((RM, _L), lambda i, j: (gp + i, 0))],
        out_specs=pl.BlockSpec((RM, dh), lambda i, j: (j * gp + i, 0)),
        out_shape=jax.ShapeDtypeStruct((2 * n_pad, dh), jnp.float32),
    )(h, degp, degp)


def _finalize(agg, degp, b2, n, n_pad, dout):
    dh = dout // 2
    RM = 2048
    g = -(-n // RM)
    gp = n_pad // RM

    def body(a0_ref, a1_ref, d0_ref, d1_ref, b_ref, o_ref):
        deg = d0_ref[:, 0:1] + d1_ref[:, 0:1] + 1.0
        dis = lax.rsqrt(deg)
        o_ref[:, 0:dh] = a0_ref[...] * dis + b_ref[:, 0:dh]
        o_ref[:, dh:dout] = a1_ref[...] * dis + b_ref[:, dh:dout]

    return pl.pallas_call(
        body,
        grid=(g,),
        in_specs=[pl.BlockSpec((RM, dh), lambda i: (i, 0)),
                  pl.BlockSpec((RM, dh), lambda i: (gp + i, 0)),
                  pl.BlockSpec((RM, _L), lambda i: (i, 0)),
                  pl.BlockSpec((RM, _L), lambda i: (gp + i, 0)),
                  pl.BlockSpec((1, dout), lambda i: (0, 0))],
        out_specs=pl.BlockSpec((RM, dout), lambda i: (i, 0)),
        out_shape=jax.ShapeDtypeStruct((n, dout), jnp.float32),
    )(agg, agg, degp, degp, b2)


def kernel(x, edge_index, W, b):
    n, _ = x.shape
    dout = W.shape[1]
    e = edge_index.shape[1]
    dh = dout // 2
    src = edge_index[0]
    dst = edge_index[1]

    n_pad = -(-n // 2048) * 2048
    chunk = _NS * 8 * _K
    e_pad = -(-e // chunk) * chunk
    pad = e_pad - e
    src_p = jnp.concatenate([src, jnp.zeros((pad,), jnp.int32)])
    trash = n + jnp.arange(pad, dtype=jnp.int32) % (n_pad - n)
    dst_p = jnp.concatenate([dst, trash])

    dstb = dst_p.reshape(e_pad // _K, _K)
    srcb = jnp.concatenate([src_p, src_p + n_pad]).reshape(
        2 * (e_pad // _K), _K)

    degp = _deg_counts(dstb, n_pad)
    h = _matmul(x, W)
    u_all = _scale(h, degp, n, n_pad, dout)
    agg = _edge_aggregate(u_all, srcb, dstb, n_pad, dh)
    return _finalize(agg, degp, b.reshape(1, dout), n, n_pad, dout)
